# baseline (speedup 1.0000x reference)
import functools

import jax
import jax.numpy as jnp
from jax import lax
from jax.experimental import pallas as pl
from jax.experimental.pallas import tpu as pltpu
from jax.experimental.pallas import tpu_sc as plsc

B = 16384
F = 14
V = 100000
D = 32
K = 7
C = 128

NC = 2
NS = 16
NW = NC * NS
ROWS_PER_W = B // NW
CHUNKS = ROWS_PER_W // C


def _body(xcat_hbm, xnum_hbm, table_hbm, wnum_hbm, bnum_hbm, wout_hbm,
          bout_hbm, out_hbm,
          xcat_v, xnum_v, idx_v, rows_v, emb_v, logit_v,
          wnum_v, bnum_v, wout_v, bout_v, sem, sem_in):
    wid = lax.axis_index("s") * NC + lax.axis_index("c")
    iota = lax.iota(jnp.int32, 16)

    pltpu.sync_copy(wnum_hbm, wnum_v)
    pltpu.sync_copy(bnum_hbm, bnum_v)
    pltpu.sync_copy(wout_hbm, wout_v)
    pltpu.sync_copy(bout_hbm, bout_v)

    wc = []
    for c in range(2):
        cc = jnp.full((16,), c, jnp.int32)
        lo = plsc.load_gather(wout_v, [iota, cc])
        hi = plsc.load_gather(wout_v, [iota + 16, cc])
        wc.append((lo, hi))

    w2 = [[None, None] for _ in range(K)]
    for k in range(K):
        r_lo = wnum_v[k, pl.ds(0, 16)]
        r_hi = wnum_v[k, pl.ds(16, 16)]
        for c in range(2):
            w2[k][c] = jnp.sum(r_lo * wc[c][0]) + jnp.sum(r_hi * wc[c][1])
    bn_lo = bnum_v[pl.ds(0, 16)]
    bn_hi = bnum_v[pl.ds(16, 16)]
    bout_vec = bout_v[pl.ds(0, 16)]
    b2 = [jnp.sum(bn_lo * wc[c][0]) + jnp.sum(bn_hi * wc[c][1]) + bout_vec[c]
          for c in range(2)]

    w_sc = [[wc[0][d // 16][d % 16], wc[1][d // 16][d % 16]]
            for d in range(D)]

    for chunk in range(CHUNKS):
        base = pl.multiple_of(wid * ROWS_PER_W + chunk * C, C)

        pltpu.sync_copy(xcat_hbm.at[pl.ds(base, C), :], xcat_v)
        pltpu.sync_copy(xnum_hbm.at[pl.ds(base, C), :], xnum_v)

        def build_idx(g, carry):
            rows16 = g * 16 + iota
            for f in range(F):
                col = jnp.full((16,), f, jnp.int32)
                v = plsc.load_gather(xcat_v, [rows16, col])
                idx_v[f, pl.ds(g * 16, 16)] = v + f * V
            return carry
        lax.fori_loop(0, C // 16, build_idx, 0)

        copies = []
        for f in range(F):
            copies.append(pltpu.async_copy(
                table_hbm.at[idx_v.at[f]], rows_v.at[f], sem))
        for cp in copies:
            cp.wait()

        def sum_row(r, carry):
            for h in range(2):
                sl = pl.ds(h * 16, 16)
                acc = rows_v[0, r, sl]
                for f in range(1, F):
                    acc = acc + rows_v[f, r, sl]
                emb_v[r, sl] = acc
            return carry
        lax.fori_loop(0, C, sum_row, 0)

        def proj(g, carry):
            rows16 = g * 16 + iota
            acc0 = jnp.zeros((16,), jnp.float32) + b2[0]
            acc1 = jnp.zeros((16,), jnp.float32) + b2[1]
            for d in range(D):
                col = jnp.full((16,), d, jnp.int32)
                ev = plsc.load_gather(emb_v, [rows16, col])
                acc0 = acc0 + ev * w_sc[d][0]
                acc1 = acc1 + ev * w_sc[d][1]
            for k in range(K):
                col = jnp.full((16,), k, jnp.int32)
                nv = plsc.load_gather(xnum_v, [rows16, col])
                acc0 = acc0 + nv * w2[k][0]
                acc1 = acc1 + nv * w2[k][1]
            plsc.store_scatter(logit_v, [rows16, jnp.full((16,), 0, jnp.int32)], acc0)
            plsc.store_scatter(logit_v, [rows16, jnp.full((16,), 1, jnp.int32)], acc1)
            return carry
        lax.fori_loop(0, C // 16, proj, 0)

        pltpu.sync_copy(logit_v, out_hbm.at[pl.ds(base, C), :])


@jax.jit
def _sc_forward(x_cat, x_num, table_flat, W_num, b_num, W_out, b_out_pad):
    mesh = plsc.VectorSubcoreMesh(core_axis_name="c", subcore_axis_name="s",
                                  num_cores=NC, num_subcores=NS)
    f = pl.kernel(
        _body,
        out_type=jax.ShapeDtypeStruct((B, 2), jnp.float32),
        mesh=mesh,
        scratch_types=[
            pltpu.VMEM((C, F), jnp.int32),
            pltpu.VMEM((C, K), jnp.float32),
            pltpu.VMEM((F, C), jnp.int32),
            pltpu.VMEM((F, C, D), jnp.float32),
            pltpu.VMEM((C, D), jnp.float32),
            pltpu.VMEM((C, 2), jnp.float32),
            pltpu.VMEM((K, D), jnp.float32),
            pltpu.VMEM((D,), jnp.float32),
            pltpu.VMEM((D, 2), jnp.float32),
            pltpu.VMEM((16,), jnp.float32),
            pltpu.SemaphoreType.DMA,
            pltpu.SemaphoreType.DMA,
        ],
        compiler_params=pltpu.CompilerParams(needs_layout_passes=False,
                                             use_tc_tiling_on_sc=False),
    )
    return f(x_cat, x_num, table_flat, W_num, b_num, W_out, b_out_pad)


def kernel(x_cat, x_num, tables, W_num, b_num, W_out, b_out, temperature):
    inv_t = (1.0 / temperature).astype(jnp.float32)
    W_out_t = (W_out * inv_t).astype(jnp.float32)
    b_out_t = (b_out * inv_t).astype(jnp.float32)
    b_out_pad = jnp.zeros((16,), jnp.float32).at[:2].set(b_out_t)
    table_flat = tables.reshape(F * V, D)
    return _sc_forward(x_cat.astype(jnp.int32), x_num.astype(jnp.float32),
                       table_flat, W_num, b_num, W_out_t, b_out_pad)
